# baseline (speedup 1.0000x reference)
import functools

import jax
import jax.numpy as jnp
from jax import lax
from jax.experimental import pallas as pl
from jax.experimental.pallas import tpu as pltpu, tpu_sc as plsc

_B, _C, _H, _W = 4, 96, 512, 512
_P = 64
_PLANE = _H * _W
_ROWS = _B * _C
_NC = 2
_L = 16
_RPT = 16
_ACTIVE = _ROWS // _RPT


def _body(xf, px_hbm, py_hbm, out_hbm,
          bidx_v, idx_v, pxb_v, pyb_v, vals_v, out_v, sem):
    wid = lax.axis_index("s") * _NC + lax.axis_index("c")

    @pl.when(wid < _ACTIVE)
    def _():
        for p in range(_P):
            bidx_v[p // 8, pl.ds((p % 8) * _L, _L)] = jnp.full(
                (_L,), p, jnp.int32)
        bcasts = []
        for j in range(8):
            sl = pl.ds(j * 8 * _L, 8 * _L)
            bcasts.append(pltpu.async_copy(px_hbm.at[bidx_v.at[j]],
                                           pxb_v.at[sl], sem))
            bcasts.append(pltpu.async_copy(py_hbm.at[bidx_v.at[j]],
                                           pyb_v.at[sl], sem))
        for cp in bcasts:
            cp.wait()
        lanes = lax.iota(jnp.int32, _L)
        row_vec = (wid * _RPT + lanes) * _PLANE
        for p in range(_P):
            sl = pl.ds(p * _L, _L)
            idx_v[p // 8, pl.ds((p % 8) * _L, _L)] = (
                row_vec + pxb_v[sl] * _W + pyb_v[sl])
        copies = [
            pltpu.async_copy(xf.at[idx_v.at[j]],
                             vals_v.at[pl.ds(j * 8 * _L, 8 * _L)], sem)
            for j in range(8)
        ]
        for cp in copies:
            cp.wait()
        acc = jnp.zeros((_L,), jnp.float32)
        for p in range(_P):
            v = vals_v[pl.ds(p * _L, _L)]
            acc = acc + v * v
        out_v[...] = acc
        pltpu.sync_copy(out_v, out_hbm.at[wid])


@functools.partial(
    pl.kernel,
    out_type=jax.ShapeDtypeStruct((_ACTIVE, _RPT), jnp.float32),
    mesh=plsc.VectorSubcoreMesh(core_axis_name="c", subcore_axis_name="s"),
    scratch_types=[
        pltpu.VMEM((8, 8 * _L), jnp.int32),
        pltpu.VMEM((8, 8 * _L), jnp.int32),
        pltpu.VMEM((_P * _L,), jnp.int32),
        pltpu.VMEM((_P * _L,), jnp.int32),
        pltpu.VMEM((_P * _L,), jnp.float32),
        pltpu.VMEM((_L,), jnp.float32),
        pltpu.SemaphoreType.DMA,
    ],
)
def _probe_kernel(xf, px_hbm, py_hbm, out_hbm, *scratch):
    _body(xf, px_hbm, py_hbm, out_hbm, *scratch)


def kernel(x, px, py):
    out = _probe_kernel(x.reshape(-1), px, py)
    return out.reshape(_B, _C)


# speedup vs baseline: 17.3391x; 17.3391x over previous
import functools

import jax
import jax.numpy as jnp
from jax import lax
from jax.experimental import pallas as pl
from jax.experimental.pallas import tpu as pltpu, tpu_sc as plsc

_B, _C, _H, _W = 4, 96, 512, 512
_P = 64
_PLANE = _H * _W
_ROWS = _B * _C
_NC = 2
_L = 16
_RPT = 16
_ACTIVE = _ROWS // _RPT
_CPR = _P // _L


def _body(xf, px_hbm, py_hbm, out_hbm,
          px_v, py_v, idx_v, tidx_v, vals_v, part_v, tpart_v, out_v,
          t_shared, sem):
    wid = lax.axis_index("s") * _NC + lax.axis_index("c")

    @pl.when(wid < _ACTIVE)
    def _():
        cpx = pltpu.async_copy(px_hbm, px_v, sem)
        cpy = pltpu.async_copy(py_hbm, py_v, sem)
        cpx.wait()
        cpy.wait()
        toffs = []
        for c in range(_CPR):
            sl = pl.ds(c * _L, _L)
            hx, wy = px_v[sl], py_v[sl]
            toffs.append((hx >> 3) * 4096 + (wy >> 7) * 1024
                         + (hx & 7) * 128 + (wy & 127))
        base0 = wid * _RPT
        for r in range(_RPT):
            row_base = (base0 + r) * _PLANE
            for c in range(_CPR):
                q = r * _CPR + c
                idx_v[q // 8, pl.ds((q % 8) * _L, _L)] = row_base + toffs[c]
        copies = [
            pltpu.async_copy(xf.at[idx_v.at[j]],
                             vals_v.at[pl.ds(j * 8 * _L, 8 * _L)], sem)
            for j in range(8)
        ]
        for cp in copies:
            cp.wait()
        for r in range(_RPT):
            acc = jnp.zeros((_L,), jnp.float32)
            for c in range(_CPR):
                v = vals_v[pl.ds((r * _CPR + c) * _L, _L)]
                acc = acc + v * v
            part_v[pl.ds(r * _L, _L)] = acc
        lanes = lax.iota(jnp.int32, _L)
        sbase = wid * (_RPT * _L)
        for m in range(_RPT):
            tidx_v[m // 8, pl.ds((m % 8) * _L, _L)] = sbase + lanes * _L + m
        for j in range(2):
            pltpu.sync_copy(part_v.at[pl.ds(j * 8 * _L, 8 * _L)],
                            t_shared.at[tidx_v.at[j]])
        pltpu.sync_copy(t_shared.at[pl.ds(sbase, _RPT * _L)], tpart_v)
        out = jnp.zeros((_L,), jnp.float32)
        for l in range(_L):
            out = out + tpart_v[pl.ds(l * _L, _L)]
        out_v[...] = out
        pltpu.sync_copy(out_v, out_hbm.at[wid])


@functools.partial(
    pl.kernel,
    out_type=jax.ShapeDtypeStruct((_ACTIVE, _RPT), jnp.float32),
    mesh=plsc.VectorSubcoreMesh(core_axis_name="c", subcore_axis_name="s"),
    scratch_types=[
        pltpu.VMEM((_P,), jnp.int32),
        pltpu.VMEM((_P,), jnp.int32),
        pltpu.VMEM((8, 8 * _L), jnp.int32),
        pltpu.VMEM((2, 8 * _L), jnp.int32),
        pltpu.VMEM((_RPT * _P,), jnp.float32),
        pltpu.VMEM((_RPT * _L,), jnp.float32),
        pltpu.VMEM((_RPT * _L,), jnp.float32),
        pltpu.VMEM((_L,), jnp.float32),
        pltpu.MemorySpace.VMEM_SHARED((_ACTIVE * _RPT * _L,), jnp.float32),
        pltpu.SemaphoreType.DMA,
    ],
)
def _probe_kernel(xf, px_hbm, py_hbm, out_hbm, *scratch):
    _body(xf, px_hbm, py_hbm, out_hbm, *scratch)


def kernel(x, px, py):
    xt = (x.reshape(_B, _C, _H // 8, 8, _W // 128, 128)
          .transpose(0, 1, 2, 4, 3, 5).reshape(-1))
    out = _probe_kernel(xt, px, py)
    return out.reshape(_B, _C)


# speedup vs baseline: 17.4501x; 1.0064x over previous
import functools

import jax
import jax.numpy as jnp
from jax import lax
from jax.experimental import pallas as pl
from jax.experimental.pallas import tpu as pltpu, tpu_sc as plsc

_B, _C, _H, _W = 4, 96, 512, 512
_P = 64
_PLANE = _H * _W
_ROWS = _B * _C
_NC = 2
_L = 16
_RPT = 16
_ACTIVE = _ROWS // _RPT
_CPR = _P // _L


def _body(xf, px_hbm, py_hbm, out_hbm,
          px_v, py_v, idx_v, tidx_v, vals_v, part_v, tpart_v, out_v,
          t_shared, sem):
    wid = lax.axis_index("s") * _NC + lax.axis_index("c")

    @pl.when(wid < _ACTIVE)
    def _():
        cpx = pltpu.async_copy(px_hbm, px_v, sem)
        cpy = pltpu.async_copy(py_hbm, py_v, sem)
        cpx.wait()
        cpy.wait()
        toffs = []
        for c in range(_CPR):
            sl = pl.ds(c * _L, _L)
            hx, wy = px_v[sl], py_v[sl]
            toffs.append((hx >> 3) * 4096 + (wy >> 7) * 1024
                         + (hx & 7) * 128 + (wy & 127))
        base0 = wid * _RPT
        for r in range(_RPT):
            row_base = (base0 + r) * _PLANE
            for c in range(_CPR):
                q = r * _CPR + c
                idx_v[q // 8, pl.ds((q % 8) * _L, _L)] = row_base + toffs[c]
        copies = [
            pltpu.async_copy(xf.at[idx_v.at[j]],
                             vals_v.at[pl.ds(j * 8 * _L, 8 * _L)], sem)
            for j in range(8)
        ]
        for cp in copies:
            cp.wait()
        for r in range(_RPT):
            acc = jnp.zeros((_L,), jnp.float32)
            for c in range(_CPR):
                v = vals_v[pl.ds((r * _CPR + c) * _L, _L)]
                acc = acc + v * v
            part_v[pl.ds(r * _L, _L)] = acc
        lanes = lax.iota(jnp.int32, _L)
        sbase = wid * (_RPT * _L)
        for m in range(_RPT):
            tidx_v[m // 8, pl.ds((m % 8) * _L, _L)] = sbase + lanes * _L + m
        scats = [
            pltpu.async_copy(part_v.at[pl.ds(j * 8 * _L, 8 * _L)],
                             t_shared.at[tidx_v.at[j]], sem)
            for j in range(2)
        ]
        for cp in scats:
            cp.wait()
        pltpu.sync_copy(t_shared.at[pl.ds(sbase, _RPT * _L)], tpart_v)
        out = jnp.zeros((_L,), jnp.float32)
        for l in range(_L):
            out = out + tpart_v[pl.ds(l * _L, _L)]
        out_v[...] = out
        pltpu.sync_copy(out_v, out_hbm.at[wid])


@functools.partial(
    pl.kernel,
    out_type=jax.ShapeDtypeStruct((_ACTIVE, _RPT), jnp.float32),
    mesh=plsc.VectorSubcoreMesh(core_axis_name="c", subcore_axis_name="s"),
    scratch_types=[
        pltpu.VMEM((_P,), jnp.int32),
        pltpu.VMEM((_P,), jnp.int32),
        pltpu.VMEM((8, 8 * _L), jnp.int32),
        pltpu.VMEM((2, 8 * _L), jnp.int32),
        pltpu.VMEM((_RPT * _P,), jnp.float32),
        pltpu.VMEM((_RPT * _L,), jnp.float32),
        pltpu.VMEM((_RPT * _L,), jnp.float32),
        pltpu.VMEM((_L,), jnp.float32),
        pltpu.MemorySpace.VMEM_SHARED((_ACTIVE * _RPT * _L,), jnp.float32),
        pltpu.SemaphoreType.DMA,
    ],
)
def _probe_kernel(xf, px_hbm, py_hbm, out_hbm, *scratch):
    _body(xf, px_hbm, py_hbm, out_hbm, *scratch)


def kernel(x, px, py):
    xt = (x.reshape(_B, _C, _H // 8, 8, _W // 128, 128)
          .transpose(0, 1, 2, 4, 3, 5).reshape(-1))
    out = _probe_kernel(xt, px, py)
    return out.reshape(_B, _C)


# speedup vs baseline: 17.5334x; 1.0048x over previous
import functools

import jax
import jax.numpy as jnp
from jax import lax
from jax.experimental import pallas as pl
from jax.experimental.pallas import tpu as pltpu, tpu_sc as plsc

_B, _C, _H, _W = 4, 96, 512, 512
_P = 64
_PLANE = _H * _W
_ROWS = _B * _C
_NC = 2
_L = 16
_RPT = 12
_ACTIVE = _ROWS // _RPT
_CPR = _P // _L
_NQ = _RPT * _CPR
_TP = _RPT * _L


def _body(xf, px_hbm, py_hbm, out_hbm,
          px_v, py_v, idx_v, tidx_v, vals_v, part_v, tpart_v, out_v,
          t_shared, sem):
    wid = lax.axis_index("s") * _NC + lax.axis_index("c")
    cpx = pltpu.async_copy(px_hbm, px_v, sem)
    cpy = pltpu.async_copy(py_hbm, py_v, sem)
    cpx.wait()
    cpy.wait()
    toffs = []
    for c in range(_CPR):
        sl = pl.ds(c * _L, _L)
        hx, wy = px_v[sl], py_v[sl]
        toffs.append((hx >> 3) * 4096 + (wy >> 7) * 1024
                     + (hx & 7) * 128 + (wy & 127))
    base0 = wid * _RPT
    for r in range(_RPT):
        row_base = (base0 + r) * _PLANE
        for c in range(_CPR):
            q = r * _CPR + c
            idx_v[q // 8, pl.ds((q % 8) * _L, _L)] = row_base + toffs[c]
    copies = [
        pltpu.async_copy(xf.at[idx_v.at[j]],
                         vals_v.at[pl.ds(j * 8 * _L, 8 * _L)], sem)
        for j in range(_NQ // 8)
    ]
    for cp in copies:
        cp.wait()
    for r in range(_RPT):
        acc = jnp.zeros((_L,), jnp.float32)
        for c in range(_CPR):
            v = vals_v[pl.ds((r * _CPR + c) * _L, _L)]
            acc = acc + v * v
        part_v[pl.ds(r * _L, _L)] = acc
    lanes = lax.iota(jnp.int32, _L)
    sbase = wid * _TP
    for m in range(_RPT):
        tidx_v[m // 6, pl.ds((m % 6) * _L, _L)] = sbase + lanes * _RPT + m
    scats = [
        pltpu.async_copy(part_v.at[pl.ds(j * 6 * _L, 6 * _L)],
                         t_shared.at[tidx_v.at[j]], sem)
        for j in range(2)
    ]
    for cp in scats:
        cp.wait()
    pltpu.sync_copy(t_shared.at[pl.ds(sbase, _TP)],
                    tpart_v.at[pl.ds(0, _TP)])
    out = jnp.zeros((_L,), jnp.float32)
    for l in range(_L):
        out = out + tpart_v[pl.ds(l * _RPT, _L)]
    out_v[...] = out
    pltpu.sync_copy(out_v, out_hbm.at[wid])


@functools.partial(
    pl.kernel,
    out_type=jax.ShapeDtypeStruct((_ACTIVE, _L), jnp.float32),
    mesh=plsc.VectorSubcoreMesh(core_axis_name="c", subcore_axis_name="s"),
    scratch_types=[
        pltpu.VMEM((_P,), jnp.int32),
        pltpu.VMEM((_P,), jnp.int32),
        pltpu.VMEM((_NQ // 8, 8 * _L), jnp.int32),
        pltpu.VMEM((2, 6 * _L), jnp.int32),
        pltpu.VMEM((_RPT * _P,), jnp.float32),
        pltpu.VMEM((_RPT * _L,), jnp.float32),
        pltpu.VMEM((_TP + _L,), jnp.float32),
        pltpu.VMEM((_L,), jnp.float32),
        pltpu.MemorySpace.VMEM_SHARED((_ACTIVE * _TP,), jnp.float32),
        pltpu.SemaphoreType.DMA,
    ],
)
def _probe_kernel(xf, px_hbm, py_hbm, out_hbm, *scratch):
    _body(xf, px_hbm, py_hbm, out_hbm, *scratch)


def kernel(x, px, py):
    xt = (x.reshape(_B, _C, _H // 8, 8, _W // 128, 128)
          .transpose(0, 1, 2, 4, 3, 5).reshape(-1))
    out = _probe_kernel(xt, px, py)
    return out[:, :_RPT].reshape(_B, _C)


# speedup vs baseline: 17.6002x; 1.0038x over previous
import functools

import jax
import jax.numpy as jnp
from jax import lax
from jax.experimental import pallas as pl
from jax.experimental.pallas import tpu as pltpu, tpu_sc as plsc

_B, _C, _H, _W = 4, 96, 512, 512
_P = 64
_PLANE = _H * _W
_ROWS = _B * _C
_NC = 2
_L = 16
_RPT = 12
_ACTIVE = _ROWS // _RPT
_CPR = _P // _L
_NQ = _RPT * _CPR
_TP = _RPT * _L


def _body(xf, px_hbm, py_hbm, out_hbm,
          px_v, py_v, idx_v, tidx_v, vals_v, part_v, tpart_v, out_v,
          t_shared, sem):
    wid = lax.axis_index("s") * _NC + lax.axis_index("c")
    cpx = pltpu.async_copy(px_hbm, px_v, sem)
    cpy = pltpu.async_copy(py_hbm, py_v, sem)
    lanes = lax.iota(jnp.int32, _L)
    sbase = wid * _TP
    for m in range(_RPT):
        tidx_v[m // 6, pl.ds((m % 6) * _L, _L)] = sbase + lanes * _RPT + m
    cpx.wait()
    cpy.wait()
    toffs = []
    for c in range(_CPR):
        sl = pl.ds(c * _L, _L)
        hx, wy = px_v[sl], py_v[sl]
        toffs.append((hx >> 3) * 4096 + (wy >> 7) * 1024
                     + (hx & 7) * 128 + (wy & 127))
    base0 = wid * _RPT
    copies = []
    for j in range(_NQ // 8):
        for k in range(8):
            q = j * 8 + k
            r, c = q // _CPR, q % _CPR
            row_base = (base0 + r) * _PLANE
            idx_v[j, pl.ds(k * _L, _L)] = row_base + toffs[c]
        copies.append(
            pltpu.async_copy(xf.at[idx_v.at[j]],
                             vals_v.at[pl.ds(j * 8 * _L, 8 * _L)], sem))
    for cp in copies:
        cp.wait()
    for r in range(_RPT):
        acc = jnp.zeros((_L,), jnp.float32)
        for c in range(_CPR):
            v = vals_v[pl.ds((r * _CPR + c) * _L, _L)]
            acc = acc + v * v
        part_v[pl.ds(r * _L, _L)] = acc
    scats = [
        pltpu.async_copy(part_v.at[pl.ds(j * 6 * _L, 6 * _L)],
                         t_shared.at[tidx_v.at[j]], sem)
        for j in range(2)
    ]
    for cp in scats:
        cp.wait()
    pltpu.sync_copy(t_shared.at[pl.ds(sbase, _TP)],
                    tpart_v.at[pl.ds(0, _TP)])
    out = jnp.zeros((_L,), jnp.float32)
    for l in range(_L):
        out = out + tpart_v[pl.ds(l * _RPT, _L)]
    out_v[...] = out
    pltpu.sync_copy(out_v, out_hbm.at[wid])


@functools.partial(
    pl.kernel,
    out_type=jax.ShapeDtypeStruct((_ACTIVE, _L), jnp.float32),
    mesh=plsc.VectorSubcoreMesh(core_axis_name="c", subcore_axis_name="s"),
    scratch_types=[
        pltpu.VMEM((_P,), jnp.int32),
        pltpu.VMEM((_P,), jnp.int32),
        pltpu.VMEM((_NQ // 8, 8 * _L), jnp.int32),
        pltpu.VMEM((2, 6 * _L), jnp.int32),
        pltpu.VMEM((_RPT * _P,), jnp.float32),
        pltpu.VMEM((_RPT * _L,), jnp.float32),
        pltpu.VMEM((_TP + _L,), jnp.float32),
        pltpu.VMEM((_L,), jnp.float32),
        pltpu.MemorySpace.VMEM_SHARED((_ACTIVE * _TP,), jnp.float32),
        pltpu.SemaphoreType.DMA,
    ],
)
def _probe_kernel(xf, px_hbm, py_hbm, out_hbm, *scratch):
    _body(xf, px_hbm, py_hbm, out_hbm, *scratch)


def kernel(x, px, py):
    xt = (x.reshape(_B, _C, _H // 8, 8, _W // 128, 128)
          .transpose(0, 1, 2, 4, 3, 5).reshape(-1))
    out = _probe_kernel(xt, px, py)
    return out[:, :_RPT].reshape(_B, _C)
